# baseline (speedup 1.0000x reference)
import functools

import jax
import jax.numpy as jnp
from jax import lax
from jax.experimental import pallas as pl
from jax.experimental.pallas import tpu as pltpu
from jax.experimental.pallas import tpu_sc as plsc

NC = 2
NS = 16
NW = NC * NS
IDXW = 128
CHUNK = 512
SUB = CHUNK // IDXW


@functools.lru_cache(maxsize=None)
def _build_gather(R, V, D):
    assert R % (NW * CHUNK) == 0
    b_per_w = R // NW
    nchunk = b_per_w // CHUNK
    assert nchunk % 2 == 0
    r_per_w = b_per_w // IDXW

    mesh = plsc.VectorSubcoreMesh(core_axis_name="c", subcore_axis_name="s")

    @functools.partial(
        pl.kernel,
        out_type=jax.ShapeDtypeStruct((R, D), jnp.float32),
        mesh=mesh,
        scratch_types=[
            pltpu.VMEM((2, SUB, IDXW), jnp.int32),
            pltpu.VMEM((2, CHUNK, D), jnp.float32),
            pltpu.SemaphoreType.DMA,
            pltpu.SemaphoreType.DMA,
        ],
        compiler_params=pltpu.CompilerParams(use_tc_tiling_on_sc=False),
    )
    def gather_kernel(table, idx2d, out, idx_v, rows_v, sem0, sem1):
        wid = lax.axis_index("s") * NC + lax.axis_index("c")
        base = wid * b_per_w
        rbase = wid * r_per_w
        sems = (sem0, sem1)

        def issue(j, slot):
            pltpu.sync_copy(
                idx2d.at[pl.ds(rbase + j * SUB, SUB)], idx_v.at[slot]
            )
            for kk in range(SUB):
                pltpu.async_copy(
                    table.at[idx_v.at[slot, kk]],
                    rows_v.at[slot, pl.ds(kk * IDXW, IDXW)],
                    sems[slot],
                )

        def drain(j, slot):
            for kk in range(SUB):
                pltpu.make_async_copy(
                    table.at[idx_v.at[slot, kk]],
                    rows_v.at[slot, pl.ds(kk * IDXW, IDXW)],
                    sems[slot],
                ).wait()
            pltpu.sync_copy(
                rows_v.at[slot],
                out.at[pl.ds(pl.multiple_of(base + j * CHUNK, CHUNK), CHUNK)],
            )

        issue(0, 0)

        def body(t, carry):
            j0 = t * 2
            issue(j0 + 1, 1)
            drain(j0, 0)

            @pl.when(j0 + 2 < nchunk)
            def _():
                issue(j0 + 2, 0)

            drain(j0 + 1, 1)
            return carry

        lax.fori_loop(0, nchunk // 2, body, None)

    return gather_kernel


def kernel(weight, input):
    R = input.size
    V, D = weight.shape
    idx2d = input.reshape(R // IDXW, IDXW).astype(jnp.int32)
    out = _build_gather(R, V, D)(weight, idx2d)
    return out.reshape(input.shape + (D,))


# speedup vs baseline: 1.0060x; 1.0060x over previous
import functools

import jax
import jax.numpy as jnp
from jax import lax
from jax.experimental import pallas as pl
from jax.experimental.pallas import tpu as pltpu
from jax.experimental.pallas import tpu_sc as plsc

NC = 2
NS = 16
NW = NC * NS
IDXW = 128
CHUNK = 256
SUBC = CHUNK // IDXW
NBUF = 4


@functools.lru_cache(maxsize=None)
def _build_gather(R, V, D):
    assert R % (NW * CHUNK * NBUF) == 0
    b_per_w = R // NW
    nchunk = b_per_w // CHUNK
    ngroup = nchunk // NBUF
    r_per_w = b_per_w // IDXW

    mesh = plsc.VectorSubcoreMesh(core_axis_name="c", subcore_axis_name="s")

    @functools.partial(
        pl.kernel,
        out_type=jax.ShapeDtypeStruct((R, D), jnp.float32),
        mesh=mesh,
        scratch_types=[
            pltpu.VMEM((r_per_w, IDXW), jnp.int32),
            pltpu.VMEM((NBUF, CHUNK, D), jnp.float32),
            [pltpu.SemaphoreType.DMA] * NBUF,
            [pltpu.SemaphoreType.DMA] * NBUF,
        ],
        compiler_params=pltpu.CompilerParams(use_tc_tiling_on_sc=False),
    )
    def gather_kernel(table, idx2d, out, idx_v, rows_v, sg, ss):
        wid = lax.axis_index("s") * NC + lax.axis_index("c")
        base = wid * b_per_w
        rbase = wid * r_per_w

        pltpu.sync_copy(idx2d.at[pl.ds(rbase, r_per_w)], idx_v)

        def gathers(s, b):
            for kk in range(SUBC):
                yield (
                    table.at[idx_v.at[s * SUBC + kk]],
                    rows_v.at[b, pl.ds(kk * IDXW, IDXW)],
                    sg[b],
                )

        def issue(s, b):
            for args in gathers(s, b):
                pltpu.async_copy(*args)

        def store_args(s, b):
            return (
                rows_v.at[b],
                out.at[pl.ds(pl.multiple_of(base + s * CHUNK, CHUNK), CHUNK)],
                ss[b],
            )

        def drain(s, b):
            for args in gathers(s, b):
                pltpu.make_async_copy(*args).wait()
            pltpu.async_copy(*store_args(s, b))

        def wait_store(s, b):
            pltpu.make_async_copy(*store_args(s, b)).wait()

        issue(0, 0)
        issue(1, 1)
        issue(2, 2)
        drain(0, 0)
        issue(3, 3)
        drain(1, 1)

        def body(t, carry):
            for b in range(NBUF):
                s = t * NBUF + b
                wait_store(s - NBUF, b)
                issue(s, b)
                b2 = (b + 2) % NBUF
                drain(s - 2, b2)
            return carry

        lax.fori_loop(1, ngroup, body, None)

        drain(nchunk - 2, (nchunk - 2) % NBUF)
        drain(nchunk - 1, (nchunk - 1) % NBUF)
        for b in range(NBUF):
            wait_store(nchunk - NBUF + b, b)

    return gather_kernel


def kernel(weight, input):
    R = input.size
    V, D = weight.shape
    idx2d = input.reshape(R // IDXW, IDXW).astype(jnp.int32)
    out = _build_gather(R, V, D)(weight, idx2d)
    return out.reshape(input.shape + (D,))


# speedup vs baseline: 1.0076x; 1.0016x over previous
import functools

import jax
import jax.numpy as jnp
from jax import lax
from jax.experimental import pallas as pl
from jax.experimental.pallas import tpu as pltpu
from jax.experimental.pallas import tpu_sc as plsc

NC = 2
NS = 16
NW = NC * NS
IDXW = 128
CHUNK = 256
SUBC = CHUNK // IDXW
NBUF = 4


@functools.lru_cache(maxsize=None)
def _build_gather(R, V, D):
    assert R % (NW * CHUNK * NBUF) == 0
    b_per_w = R // NW
    nchunk = b_per_w // CHUNK
    ngroup = nchunk // NBUF
    r_per_w = b_per_w // IDXW

    mesh = plsc.VectorSubcoreMesh(core_axis_name="c", subcore_axis_name="s")

    @functools.partial(
        pl.kernel,
        out_type=jax.ShapeDtypeStruct((R, D), jnp.float32),
        mesh=mesh,
        scratch_types=[
            pltpu.VMEM((r_per_w, IDXW), jnp.int32),
            pltpu.VMEM((NBUF, CHUNK, D), jnp.float32),
            [pltpu.SemaphoreType.DMA] * NBUF,
            [pltpu.SemaphoreType.DMA] * NBUF,
        ],
        compiler_params=pltpu.CompilerParams(
            use_tc_tiling_on_sc=False,
            disable_bounds_checks=True,
            disable_semaphore_checks=True,
            skip_device_barrier=True,
        ),
    )
    def gather_kernel(table, idx2d, out, idx_v, rows_v, sg, ss):
        wid = lax.axis_index("s") * NC + lax.axis_index("c")
        base = wid * b_per_w
        rbase = wid * r_per_w

        pltpu.sync_copy(idx2d.at[pl.ds(rbase, r_per_w)], idx_v)

        def gathers(s, b):
            for kk in range(SUBC):
                yield (
                    table.at[idx_v.at[s * SUBC + kk]],
                    rows_v.at[b, pl.ds(kk * IDXW, IDXW)],
                    sg[b],
                )

        def issue(s, b):
            for args in gathers(s, b):
                pltpu.async_copy(*args)

        def store_args(s, b):
            return (
                rows_v.at[b],
                out.at[pl.ds(pl.multiple_of(base + s * CHUNK, CHUNK), CHUNK)],
                ss[b],
            )

        def drain(s, b):
            for args in gathers(s, b):
                pltpu.make_async_copy(*args).wait()
            pltpu.async_copy(*store_args(s, b))

        def wait_store(s, b):
            pltpu.make_async_copy(*store_args(s, b)).wait()

        issue(0, 0)
        issue(1, 1)
        issue(2, 2)
        drain(0, 0)
        issue(3, 3)
        drain(1, 1)

        def body(t, carry):
            for b in range(NBUF):
                s = t * NBUF + b
                wait_store(s - NBUF, b)
                issue(s, b)
                b2 = (b + 2) % NBUF
                drain(s - 2, b2)
            return carry

        lax.fori_loop(1, ngroup, body, None)

        drain(nchunk - 2, (nchunk - 2) % NBUF)
        drain(nchunk - 1, (nchunk - 1) % NBUF)
        for b in range(NBUF):
            wait_store(nchunk - NBUF + b, b)

    return gather_kernel


def kernel(weight, input):
    R = input.size
    V, D = weight.shape
    idx2d = input.reshape(R // IDXW, IDXW).astype(jnp.int32)
    out = _build_gather(R, V, D)(weight, idx2d)
    return out.reshape(input.shape + (D,))
